# baseline (speedup 1.0000x reference)
import functools

import jax
import jax.numpy as jnp
from jax import lax
from jax.experimental import pallas as pl
from jax.experimental.pallas import tpu as pltpu
from jax.experimental.pallas import tpu_sc as plsc

N = 100000
H = 16
NACC = 100096
STRIPE = NACC // 16
ZCH = STRIPE // 8
ZCH2 = STRIPE // 16
NC, NS, LN = 2, 16, 16
W = NC * NS
KJ = 8
KJP = 4
GROUP_E = W * 128 * KJ
GROUP_P = W * 128 * KJP
CORE0_FRAC = 0.4

BN = 2000
GRID = N // BN

_MESH = plsc.VectorSubcoreMesh(core_axis_name="c", subcore_axis_name="s")
_SC_PARAMS = pltpu.CompilerParams(use_tc_tiling_on_sc=False)


def _pad_edges(src, dst):
    e = src.shape[0]
    e2 = ((e + GROUP_P - 1) // GROUP_P) * GROUP_P
    pad = e2 - e
    src = jnp.concatenate([src, jnp.zeros((pad,), jnp.int32)])
    dst = jnp.concatenate([dst, jnp.full((pad,), DUMMY, jnp.int32)])
    tp = e2 // (16 * KJP * 128)
    g0 = 2 * int(round(tp * CORE0_FRAC / 2))
    g0 = min(max(g0, 2), tp - 2)
    return src.reshape(-1, 128), dst.reshape(-1, 128), (g0, tp - g0)


def _make_propagate(groups_list):
    nsets = len(groups_list)

    def body(*refs):
        ins = refs[: 3 * nsets]
        out0, out1 = refs[3 * nsets: 3 * nsets + 2]
        (sidx0, didx0, rbuf0, sidx1, didx1, rbuf1, zbuf, acc,
         sem0, sem1) = refs[3 * nsets + 2:]
        c = lax.axis_index("c")
        s = lax.axis_index("s")
        wid = s * NC + c

        def zb(i, carry):
            zbuf[i] = jnp.zeros((H,), jnp.float32)
            return carry

        lax.fori_loop(0, ZCH2, zb, 0)
        for k in range(16):
            pltpu.sync_copy(zbuf, acc.at[pl.ds(s * STRIPE + k * ZCH2, ZCH2)])
        plsc.subcore_barrier()

        for t in range(nsets):
            srcm, dstm, xh = ins[3 * t: 3 * t + 3]
            G0, G1 = groups_list[t]
            Gc = jnp.where(c == 0, G0, G1)
            wbase = jnp.where(c == 0, s * (G0 * KJP),
                              16 * G0 * KJP + s * (G1 * KJP))

            def fire(roff, sidx, didx, rbuf, sem, srcm=srcm, dstm=dstm,
                     xh=xh):
                pltpu.sync_copy(srcm.at[pl.ds(roff, KJP)], sidx)
                pltpu.sync_copy(dstm.at[pl.ds(roff, KJP)], didx)
                for j in range(KJP):
                    pltpu.async_copy(xh.at[sidx.at[j]], rbuf.at[j], sem)

            def wait_g(sidx, rbuf, sem, xh=xh):
                for j in range(KJP):
                    pltpu.make_async_copy(
                        xh.at[sidx.at[j]], rbuf.at[j], sem).wait()

            def scat(didx, rbuf):
                for j in range(KJP):
                    pltpu.sync_copy(rbuf.at[j], acc.at[didx.at[j]], add=True)

            fire(wbase, sidx0, didx0, rbuf0, sem0)

            def dbl(i, carry, wbase=wbase, Gc=Gc, fire=fire, wait_g=wait_g,
                    scat=scat):
                g0 = 2 * i
                fire(wbase + (g0 + 1) * KJP, sidx1, didx1, rbuf1, sem1)
                wait_g(sidx0, rbuf0, sem0)
                scat(didx0, rbuf0)
                fire(wbase + jnp.minimum(g0 + 2, Gc - 1) * KJP,
                     sidx0, didx0, rbuf0, sem0)
                wait_g(sidx1, rbuf1, sem1)
                scat(didx1, rbuf1)
                return carry

            lax.fori_loop(0, Gc // 2, dbl, 0)
            wait_g(sidx0, rbuf0, sem0)

        plsc.subcore_barrier()

        @pl.when(c == 0)
        def _():
            pltpu.sync_copy(acc.at[pl.ds(s * STRIPE, STRIPE)],
                            out0.at[pl.ds(s * STRIPE, STRIPE)])

        @pl.when(c == 1)
        def _():
            pltpu.sync_copy(acc.at[pl.ds(s * STRIPE, STRIPE)],
                            out1.at[pl.ds(s * STRIPE, STRIPE)])

    return pl.kernel(
        body,
        out_type=[jax.ShapeDtypeStruct((NACC, H), jnp.float32)] * 2,
        mesh=_MESH,
        compiler_params=_SC_PARAMS,
        scratch_types=[
            pltpu.VMEM((KJP, 128), jnp.int32),
            pltpu.VMEM((KJP, 128), jnp.int32),
            pltpu.VMEM((KJP, 128, H), jnp.float32),
            pltpu.VMEM((KJP, 128), jnp.int32),
            pltpu.VMEM((KJP, 128), jnp.int32),
            pltpu.VMEM((KJP, 128, H), jnp.float32),
            pltpu.VMEM((ZCH2, H), jnp.float32),
            pltpu.VMEM_SHARED((NACC, H), jnp.float32),
            pltpu.SemaphoreType.DMA,
            pltpu.SemaphoreType.DMA,
        ],
    )


def _make_deg(groups_list):
    nsets = len(groups_list)

    def body(*refs):
        ins = refs[:nsets]
        out0, out1 = refs[nsets: nsets + 2]
        cidx, ones_v, degv, ebuf, deg_sh = refs[nsets + 2:]
        c = lax.axis_index("c")
        s = lax.axis_index("s")
        for k in range(8):
            ones_v[pl.ds(k * LN, LN)] = jnp.ones((LN,), jnp.float32)

        def zb(i, carry):
            degv[pl.ds(i * LN, LN)] = jnp.zeros((LN,), jnp.float32)
            return carry

        lax.fori_loop(0, STRIPE // LN, zb, 0)
        pltpu.sync_copy(degv, deg_sh.at[pl.ds(s * STRIPE, STRIPE)])
        plsc.subcore_barrier()

        for t in range(nsets):
            dstm = ins[t]
            G0, G1 = groups_list[t]
            Gc = jnp.where(c == 0, G0, G1)
            wbase = jnp.where(c == 0, s * (G0 * KJP),
                              16 * G0 * KJP + s * (G1 * KJP))

            def grp(g, carry, dstm=dstm, wbase=wbase):
                roff = wbase + g * KJP
                pltpu.sync_copy(dstm.at[pl.ds(roff, KJP)], cidx)
                for j in range(KJP):
                    pltpu.sync_copy(ones_v, deg_sh.at[cidx.at[j]], add=True)
                return carry

            lax.fori_loop(0, Gc, grp, 0)

        plsc.subcore_barrier()

        pltpu.sync_copy(deg_sh.at[pl.ds(s * STRIPE, STRIPE)], degv)

        def exp(i, carry):
            v = degv[pl.ds(i * LN, LN)]
            for l in range(LN):
                ebuf[i * LN + l] = jnp.full((H,), v[l])
            return carry

        lax.fori_loop(0, STRIPE // LN, exp, 0)

        @pl.when(c == 0)
        def _():
            pltpu.sync_copy(ebuf, out0.at[pl.ds(s * STRIPE, STRIPE)])

        @pl.when(c == 1)
        def _():
            pltpu.sync_copy(ebuf, out1.at[pl.ds(s * STRIPE, STRIPE)])

    return pl.kernel(
        body,
        out_type=[jax.ShapeDtypeStruct((NACC, H), jnp.float32)] * 2,
        mesh=_MESH,
        compiler_params=_SC_PARAMS,
        scratch_types=[
            pltpu.VMEM((KJP, 128), jnp.int32),
            pltpu.VMEM((128,), jnp.float32),
            pltpu.VMEM((STRIPE,), jnp.float32),
            pltpu.VMEM((STRIPE, H), jnp.float32),
            pltpu.VMEM_SHARED((NACC,), jnp.float32),
        ],
    )


def _pairs_gather(pairs_m, a0, a1, dv, s2):
    def body(pm, t0, t1, t2, t3, o0, o1, o2, o3, pidx, gbuf, sem):
        c = lax.axis_index("c")
        s = lax.axis_index("s")
        wid = s * NC + c
        pltpu.sync_copy(pm.at[pl.ds(wid * 2, 2)], pidx)
        for th, out in ((t0, o0), (t1, o1), (t2, o2), (t3, o3)):
            descs = [
                pltpu.async_copy(th.at[pidx.at[j]], gbuf.at[j], sem)
                for j in range(2)
            ]
            for d in descs:
                d.wait()
            pltpu.sync_copy(gbuf, out.at[pl.ds(wid * 2, 2)])

    f = pl.kernel(
        body,
        out_type=[jax.ShapeDtypeStruct((64, 128, H), jnp.float32)] * 4,
        mesh=_MESH,
        compiler_params=_SC_PARAMS,
        scratch_types=[
            pltpu.VMEM((2, 128), jnp.int32),
            pltpu.VMEM((2, 128, H), jnp.float32),
            pltpu.SemaphoreType.DMA,
        ],
    )
    return f(pairs_m, a0, a1, dv, s2)


NAP = NACC // 8
BR = NAP // 4
DUMMY = N


def _pk_spec():
    return pl.BlockSpec((BR, 128), lambda i: (i, 0))


def _full_spec(shape):
    nd = len(shape)
    return pl.BlockSpec(shape, lambda i: (0,) * nd)


def _preA_tc(xd4, xp4, Wd2p8, Wp2d8, Wp2p8, Wsd8, Wsp8):
    def body(xd_r, xp_r, wd2p, wp2d, wp2p, wsd, wsp,
             d2p_o, p2d_o, p2p_o, sd_o, sp_o):
        a = pl.program_id(1)
        xd = xd_r[:, 0, 0, :]
        xp = xp_r[:, 0, 0, :]
        dot = functools.partial(jnp.dot, preferred_element_type=jnp.float32)
        outs = ((d2p_o, xd, wd2p), (p2d_o, xp, wp2d), (p2p_o, xp, wp2p),
                (sd_o, xd, wsd), (sp_o, xp, wsp))

        @pl.when(a == 0)
        def _():
            for o, x, w in outs:
                o[...] = dot(x, w[0])

        @pl.when(a != 0)
        def _():
            for o, x, w in outs:
                o[...] += dot(x, w[0])

    x4spec = pl.BlockSpec((BR, 1, 1, 128), lambda i, a: (i, a, 0, 0))
    wspec = pl.BlockSpec((1, 128, 128), lambda i, a: (a, 0, 0))
    pspec = pl.BlockSpec((BR, 128), lambda i, a: (i, 0))
    return pl.pallas_call(
        body,
        grid=(4, 8),
        in_specs=[x4spec, x4spec, wspec, wspec, wspec, wspec, wspec],
        out_specs=[pspec] * 5,
        out_shape=[jax.ShapeDtypeStruct((NAP, 128), jnp.float32)] * 5,
    )(xd4, xp4, Wd2p8, Wp2d8, Wp2p8, Wsd8, Wsp8)


def _preB_tc(dg0p, dg1p, y1, y2, y3, y4, y5, isd_p, b1, b2, b3, b4, b5):
    def body(d0_r, d1_r, y1_r, y2_r, y3_r, y4_r, y5_r, isd_r,
             b1_r, b2_r, b3_r, b4_r, b5_r,
             d2p_o, p2d_o, p2p_o, sd_o, sp_o, dinv_o):
        deg = d0_r[...] + d1_r[...]
        dinv = jnp.where(deg > 0, lax.rsqrt(jnp.maximum(deg, 1e-30)), 0.0)
        md = isd_r[...]
        mp = 1.0 - md
        d2p_o[...] = (y1_r[...] + b1_r[...]) * md * dinv
        p2d_o[...] = (y2_r[...] + b2_r[...]) * mp * dinv
        p2p_o[...] = (y3_r[...] + b3_r[...]) * mp * dinv
        sd_o[...] = (y4_r[...] + b4_r[...]) * md
        sp_o[...] = (y5_r[...] + b5_r[...]) * mp
        dinv_o[...] = dinv

    pspec = _pk_spec()
    bspec = _full_spec((1, 128))
    return pl.pallas_call(
        body,
        grid=(4,),
        in_specs=[pspec] * 8 + [bspec] * 5,
        out_specs=[pspec] * 6,
        out_shape=[jax.ShapeDtypeStruct((NAP, 128), jnp.float32)] * 6,
    )(dg0p, dg1p, y1, y2, y3, y4, y5, isd_p, b1, b2, b3, b4, b5)


def _midA_tc(pacc0, pacc1, dinv_p, sp, isd_p, W2p2d_big, b2p2d_p):
    def body(p0_r, p1_r, dinv_r, sp_r, isd_r, wp2d, bp2d, p2d2_o):
        dinv = dinv_r[...]
        mp = 1.0 - isd_r[...]
        hp = jnp.maximum(dinv * (p0_r[...] + p1_r[...]) + sp_r[...], 0.0)
        y = jnp.dot(hp, wp2d[...], preferred_element_type=jnp.float32)
        p2d2_o[...] = (y + bp2d[...]) * mp * dinv

    pspec = _pk_spec()
    return pl.pallas_call(
        body,
        grid=(4,),
        in_specs=[pspec, pspec, pspec, pspec, pspec,
                  _full_spec((128, 128)), _full_spec((1, 128))],
        out_specs=pspec,
        out_shape=jax.ShapeDtypeStruct((NAP, 128), jnp.float32),
    )(pacc0, pacc1, dinv_p, sp, isd_p, W2p2d_big, b2p2d_p)


def _midB_tc(dacc0, dacc1, dinv_p, sd, isd_p, W2sd_big, b2sd_p):
    def body(d0_r, d1_r, dinv_r, sd_r, isd_r, wsd, bsd, sd2_o):
        dinv = dinv_r[...]
        md = isd_r[...]
        hd = jnp.maximum(dinv * (d0_r[...] + d1_r[...]) + sd_r[...], 0.0)
        y = jnp.dot(hd, wsd[...], preferred_element_type=jnp.float32)
        sd2_o[...] = (y + bsd[...]) * md

    pspec = _pk_spec()
    return pl.pallas_call(
        body,
        grid=(4,),
        in_specs=[pspec, pspec, pspec, pspec, pspec,
                  _full_spec((128, 128)), _full_spec((1, 128))],
        out_specs=pspec,
        out_shape=jax.ShapeDtypeStruct((NAP, 128), jnp.float32),
    )(dacc0, dacc1, dinv_p, sd, isd_p, W2sd_big, b2sd_p)


def _head_tc(ga0, ga1, gdv, gs2, clf, predictor, nb):
    def body(a0_r, a1_r, dv_r, s2_r, cl_r, p_r, out_r):
        hd = dv_r[...] * (a0_r[...] + a1_r[...]) + s2_r[...]
        h1 = hd[0:nb, :]
        h2 = hd[nb:2 * nb, :]
        cl = cl_r[...]
        acc = jnp.zeros((nb, 1), jnp.float32)
        for c in range(4):
            t = jnp.dot(h1, p_r[c], preferred_element_type=jnp.float32)
            s = jnp.sum(t * h2, axis=1, keepdims=True)
            acc = acc + jnp.where(cl == c, s, 0.0)
        out_r[...] = acc

    gspec = _full_spec((2 * nb, H))
    return pl.pallas_call(
        body,
        grid=(1,),
        in_specs=[gspec, gspec, gspec, gspec, _full_spec((nb, 1)),
                  _full_spec((4, H, H))],
        out_specs=_full_spec((nb, 1)),
        out_shape=jax.ShapeDtypeStruct((nb, 1), jnp.float32),
    )(ga0, ga1, gdv, gs2, clf, predictor)


def kernel(x_drugs, x_prots, is_drug, ppi_edge_idx, dpi_edge_idx, drug_pairs,
           cell_lines, predictor,
           W1_d2p, b1_d2p, W1_p2d, b1_p2d, W1_p2p, b1_p2p, W1_sd, b1_sd,
           W1_sp, b1_sp,
           W2_d2p, b2_d2p, W2_p2d, b2_p2d, W2_p2p, b2_p2p, W2_sd, b2_sd,
           W2_sp, b2_sp):
    ppi = ppi_edge_idx.astype(jnp.int32)
    dpi = dpi_edge_idx.astype(jnp.int32)

    dpi_s, dpi_d, dpi_g = _pad_edges(dpi[0], dpi[1])
    pdi_s, pdi_d, pdi_g = _pad_edges(dpi[1], dpi[0])
    ppi_s, ppi_d, ppi_g = _pad_edges(ppi[0], ppi[1])

    dg0, dg1 = _make_deg([dpi_g, pdi_g, ppi_g])(dpi_d, pdi_d, ppi_d)

    xd4 = jnp.pad(x_drugs, ((0, NACC - N), (0, 0))).reshape(NAP, 8, 1, 128)
    xp4 = jnp.pad(x_prots, ((0, NACC - N), (0, 0))).reshape(NAP, 8, 1, 128)
    S8 = jnp.eye(128, dtype=jnp.float32).reshape(8, H, 128)

    def w8(w):
        return jnp.einsum('kj,ajl->akl', w, S8)

    y1, y2, y3, y4, y5 = _preA_tc(
        xd4, xp4, w8(W1_d2p), w8(W1_p2d), w8(W1_p2p), w8(W1_sd), w8(W1_sp))

    isd_p = jnp.pad(jnp.repeat(is_drug, H),
                    (0, (NACC - N) * H)).reshape(NAP, 128)

    def bt(b):
        return jnp.tile(b, 8).reshape(1, 128)

    def quarter(a):
        return a.reshape(NAP, 128)

    d2p_s, p2d_s, p2p_s, sd, sp, dinv_p = _preB_tc(
        quarter(dg0), quarter(dg1), y1, y2, y3, y4, y5, isd_p,
        bt(b1_d2p), bt(b1_p2d), bt(b1_p2p), bt(b1_sd), bt(b1_sp))

    prop2 = _make_propagate([pdi_g])
    prop1 = _make_propagate([dpi_g, ppi_g])

    def lin(a):
        return a.reshape(NACC, H)

    pacc0, pacc1 = prop1(dpi_s, dpi_d, lin(d2p_s), ppi_s, ppi_d, lin(p2p_s))
    dacc0, dacc1 = prop2(pdi_s, pdi_d, lin(p2d_s))

    eye8 = jnp.eye(8, dtype=jnp.float32)
    p2d2_s = _midA_tc(quarter(pacc0), quarter(pacc1), dinv_p, sp, isd_p,
                      jnp.kron(eye8, W2_p2d), bt(b2_p2d))
    sd2 = _midB_tc(quarter(dacc0), quarter(dacc1), dinv_p, sd, isd_p,
                   jnp.kron(eye8, W2_sd), bt(b2_sd))

    acc20, acc21 = prop2(pdi_s, pdi_d, lin(p2d2_s))

    nb = drug_pairs.shape[0]
    pairs_m = jnp.concatenate(
        [drug_pairs[:, 0].astype(jnp.int32),
         drug_pairs[:, 1].astype(jnp.int32)]).reshape(64, 128)
    ga0, ga1, gdv, gs2 = _pairs_gather(pairs_m, acc20, acc21,
                                       lin(dinv_p), lin(sd2))

    clf = cell_lines.astype(jnp.float32).reshape(nb, 1)
    out = _head_tc(ga0.reshape(2 * nb, H), ga1.reshape(2 * nb, H),
                   gdv.reshape(2 * nb, H), gs2.reshape(2 * nb, H),
                   clf, predictor, nb)
    return out.reshape(nb)


# speedup vs baseline: 1.0993x; 1.0993x over previous
import functools

import jax
import jax.numpy as jnp
from jax import lax
from jax.experimental import pallas as pl
from jax.experimental.pallas import tpu as pltpu
from jax.experimental.pallas import tpu_sc as plsc

N = 100000
H = 16
NACC = 100096
STRIPE = NACC // 16
ZCH = STRIPE // 8
ZCH2 = STRIPE // 16
NC, NS, LN = 2, 16, 16
W = NC * NS
KJ = 8
KJP = 4
GROUP_E = W * 128 * KJ
GROUP_P = W * 128 * KJP
CORE0_FRAC = 0.65

BN = 2000
GRID = N // BN

_MESH = plsc.VectorSubcoreMesh(core_axis_name="c", subcore_axis_name="s")
_SC_PARAMS = pltpu.CompilerParams(use_tc_tiling_on_sc=False)


def _pad_edges(src, dst):
    e = src.shape[0]
    e2 = ((e + GROUP_P - 1) // GROUP_P) * GROUP_P
    pad = e2 - e
    src = jnp.concatenate([src, jnp.zeros((pad,), jnp.int32)])
    dst = jnp.concatenate([dst, jnp.full((pad,), DUMMY, jnp.int32)])
    tp = e2 // (16 * KJP * 128)
    g0 = 2 * int(round(tp * CORE0_FRAC / 2))
    g0 = min(max(g0, 2), tp - 2)
    return src.reshape(-1, 128), dst.reshape(-1, 128), (g0, tp - g0)


def _make_propagate(groups_list):
    nsets = len(groups_list)

    def body(*refs):
        ins = refs[: 3 * nsets]
        out0, out1 = refs[3 * nsets: 3 * nsets + 2]
        (sidx0, didx0, rbuf0, sidx1, didx1, rbuf1, zbuf, acc,
         sem0, sem1) = refs[3 * nsets + 2:]
        c = lax.axis_index("c")
        s = lax.axis_index("s")
        wid = s * NC + c

        def zb(i, carry):
            zbuf[i] = jnp.zeros((H,), jnp.float32)
            return carry

        lax.fori_loop(0, ZCH2, zb, 0)
        for k in range(16):
            pltpu.sync_copy(zbuf, acc.at[pl.ds(s * STRIPE + k * ZCH2, ZCH2)])
        plsc.subcore_barrier()

        for t in range(nsets):
            srcm, dstm, xh = ins[3 * t: 3 * t + 3]
            G0, G1 = groups_list[t]
            Gc = jnp.where(c == 0, G0, G1)
            wbase = jnp.where(c == 0, s * (G0 * KJP),
                              16 * G0 * KJP + s * (G1 * KJP))

            def fire(roff, sidx, didx, rbuf, sem, srcm=srcm, dstm=dstm,
                     xh=xh):
                pltpu.sync_copy(srcm.at[pl.ds(roff, KJP)], sidx)
                pltpu.sync_copy(dstm.at[pl.ds(roff, KJP)], didx)
                for j in range(KJP):
                    pltpu.async_copy(xh.at[sidx.at[j]], rbuf.at[j], sem)

            def wait_g(sidx, rbuf, sem, xh=xh):
                for j in range(KJP):
                    pltpu.make_async_copy(
                        xh.at[sidx.at[j]], rbuf.at[j], sem).wait()

            def scat(didx, rbuf):
                for j in range(KJP):
                    pltpu.sync_copy(rbuf.at[j], acc.at[didx.at[j]], add=True)

            fire(wbase, sidx0, didx0, rbuf0, sem0)

            def dbl(i, carry, wbase=wbase, Gc=Gc, fire=fire, wait_g=wait_g,
                    scat=scat):
                g0 = 2 * i
                fire(wbase + (g0 + 1) * KJP, sidx1, didx1, rbuf1, sem1)
                wait_g(sidx0, rbuf0, sem0)
                scat(didx0, rbuf0)
                fire(wbase + jnp.minimum(g0 + 2, Gc - 1) * KJP,
                     sidx0, didx0, rbuf0, sem0)
                wait_g(sidx1, rbuf1, sem1)
                scat(didx1, rbuf1)
                return carry

            lax.fori_loop(0, Gc // 2, dbl, 0)
            wait_g(sidx0, rbuf0, sem0)

        plsc.subcore_barrier()

        @pl.when(c == 0)
        def _():
            pltpu.sync_copy(acc.at[pl.ds(s * STRIPE, STRIPE)],
                            out0.at[pl.ds(s * STRIPE, STRIPE)])

        @pl.when(c == 1)
        def _():
            pltpu.sync_copy(acc.at[pl.ds(s * STRIPE, STRIPE)],
                            out1.at[pl.ds(s * STRIPE, STRIPE)])

    return pl.kernel(
        body,
        out_type=[jax.ShapeDtypeStruct((NACC, H), jnp.float32)] * 2,
        mesh=_MESH,
        compiler_params=_SC_PARAMS,
        scratch_types=[
            pltpu.VMEM((KJP, 128), jnp.int32),
            pltpu.VMEM((KJP, 128), jnp.int32),
            pltpu.VMEM((KJP, 128, H), jnp.float32),
            pltpu.VMEM((KJP, 128), jnp.int32),
            pltpu.VMEM((KJP, 128), jnp.int32),
            pltpu.VMEM((KJP, 128, H), jnp.float32),
            pltpu.VMEM((ZCH2, H), jnp.float32),
            pltpu.VMEM_SHARED((NACC, H), jnp.float32),
            pltpu.SemaphoreType.DMA,
            pltpu.SemaphoreType.DMA,
        ],
    )


def _make_deg(groups_list):
    nsets = len(groups_list)

    def body(*refs):
        ins = refs[:nsets]
        out0, out1 = refs[nsets: nsets + 2]
        cidx, ones_v, degv, ebuf, deg_sh = refs[nsets + 2:]
        c = lax.axis_index("c")
        s = lax.axis_index("s")
        for k in range(8):
            ones_v[pl.ds(k * LN, LN)] = jnp.ones((LN,), jnp.float32)

        def zb(i, carry):
            degv[pl.ds(i * LN, LN)] = jnp.zeros((LN,), jnp.float32)
            return carry

        lax.fori_loop(0, STRIPE // LN, zb, 0)
        pltpu.sync_copy(degv, deg_sh.at[pl.ds(s * STRIPE, STRIPE)])
        plsc.subcore_barrier()

        for t in range(nsets):
            dstm = ins[t]
            G0, G1 = groups_list[t]
            Gc = jnp.where(c == 0, G0, G1)
            wbase = jnp.where(c == 0, s * (G0 * KJP),
                              16 * G0 * KJP + s * (G1 * KJP))

            def grp(g, carry, dstm=dstm, wbase=wbase):
                roff = wbase + g * KJP
                pltpu.sync_copy(dstm.at[pl.ds(roff, KJP)], cidx)
                for j in range(KJP):
                    pltpu.sync_copy(ones_v, deg_sh.at[cidx.at[j]], add=True)
                return carry

            lax.fori_loop(0, Gc, grp, 0)

        plsc.subcore_barrier()

        pltpu.sync_copy(deg_sh.at[pl.ds(s * STRIPE, STRIPE)], degv)

        def exp(i, carry):
            v = degv[pl.ds(i * LN, LN)]
            for l in range(LN):
                ebuf[i * LN + l] = jnp.full((H,), v[l])
            return carry

        lax.fori_loop(0, STRIPE // LN, exp, 0)

        @pl.when(c == 0)
        def _():
            pltpu.sync_copy(ebuf, out0.at[pl.ds(s * STRIPE, STRIPE)])

        @pl.when(c == 1)
        def _():
            pltpu.sync_copy(ebuf, out1.at[pl.ds(s * STRIPE, STRIPE)])

    return pl.kernel(
        body,
        out_type=[jax.ShapeDtypeStruct((NACC, H), jnp.float32)] * 2,
        mesh=_MESH,
        compiler_params=_SC_PARAMS,
        scratch_types=[
            pltpu.VMEM((KJP, 128), jnp.int32),
            pltpu.VMEM((128,), jnp.float32),
            pltpu.VMEM((STRIPE,), jnp.float32),
            pltpu.VMEM((STRIPE, H), jnp.float32),
            pltpu.VMEM_SHARED((NACC,), jnp.float32),
        ],
    )


def _pairs_gather(pairs_m, a0, a1, dv, s2):
    def body(pm, t0, t1, t2, t3, o0, o1, o2, o3, pidx, gbuf, sem):
        c = lax.axis_index("c")
        s = lax.axis_index("s")
        wid = s * NC + c
        pltpu.sync_copy(pm.at[pl.ds(wid * 2, 2)], pidx)
        for th, out in ((t0, o0), (t1, o1), (t2, o2), (t3, o3)):
            descs = [
                pltpu.async_copy(th.at[pidx.at[j]], gbuf.at[j], sem)
                for j in range(2)
            ]
            for d in descs:
                d.wait()
            pltpu.sync_copy(gbuf, out.at[pl.ds(wid * 2, 2)])

    f = pl.kernel(
        body,
        out_type=[jax.ShapeDtypeStruct((64, 128, H), jnp.float32)] * 4,
        mesh=_MESH,
        compiler_params=_SC_PARAMS,
        scratch_types=[
            pltpu.VMEM((2, 128), jnp.int32),
            pltpu.VMEM((2, 128, H), jnp.float32),
            pltpu.SemaphoreType.DMA,
        ],
    )
    return f(pairs_m, a0, a1, dv, s2)


NAP = NACC // 8
BR = NAP // 4
DUMMY = N


def _pk_spec():
    return pl.BlockSpec((BR, 128), lambda i: (i, 0))


def _full_spec(shape):
    nd = len(shape)
    return pl.BlockSpec(shape, lambda i: (0,) * nd)


def _preA_tc(xd4, xp4, Wd2p8, Wp2d8, Wp2p8, Wsd8, Wsp8):
    def body(xd_r, xp_r, wd2p, wp2d, wp2p, wsd, wsp,
             d2p_o, p2d_o, p2p_o, sd_o, sp_o):
        a = pl.program_id(1)
        xd = xd_r[:, 0, 0, :]
        xp = xp_r[:, 0, 0, :]
        dot = functools.partial(jnp.dot, preferred_element_type=jnp.float32)
        outs = ((d2p_o, xd, wd2p), (p2d_o, xp, wp2d), (p2p_o, xp, wp2p),
                (sd_o, xd, wsd), (sp_o, xp, wsp))

        @pl.when(a == 0)
        def _():
            for o, x, w in outs:
                o[...] = dot(x, w[0])

        @pl.when(a != 0)
        def _():
            for o, x, w in outs:
                o[...] += dot(x, w[0])

    x4spec = pl.BlockSpec((BR, 1, 1, 128), lambda i, a: (i, a, 0, 0))
    wspec = pl.BlockSpec((1, 128, 128), lambda i, a: (a, 0, 0))
    pspec = pl.BlockSpec((BR, 128), lambda i, a: (i, 0))
    return pl.pallas_call(
        body,
        grid=(4, 8),
        in_specs=[x4spec, x4spec, wspec, wspec, wspec, wspec, wspec],
        out_specs=[pspec] * 5,
        out_shape=[jax.ShapeDtypeStruct((NAP, 128), jnp.float32)] * 5,
    )(xd4, xp4, Wd2p8, Wp2d8, Wp2p8, Wsd8, Wsp8)


def _preB_tc(dg0p, dg1p, y1, y2, y3, y4, y5, isd_p, b1, b2, b3, b4, b5):
    def body(d0_r, d1_r, y1_r, y2_r, y3_r, y4_r, y5_r, isd_r,
             b1_r, b2_r, b3_r, b4_r, b5_r,
             d2p_o, p2d_o, p2p_o, sd_o, sp_o, dinv_o):
        deg = d0_r[...] + d1_r[...]
        dinv = jnp.where(deg > 0, lax.rsqrt(jnp.maximum(deg, 1e-30)), 0.0)
        md = isd_r[...]
        mp = 1.0 - md
        d2p_o[...] = (y1_r[...] + b1_r[...]) * md * dinv
        p2d_o[...] = (y2_r[...] + b2_r[...]) * mp * dinv
        p2p_o[...] = (y3_r[...] + b3_r[...]) * mp * dinv
        sd_o[...] = (y4_r[...] + b4_r[...]) * md
        sp_o[...] = (y5_r[...] + b5_r[...]) * mp
        dinv_o[...] = dinv

    pspec = _pk_spec()
    bspec = _full_spec((1, 128))
    return pl.pallas_call(
        body,
        grid=(4,),
        in_specs=[pspec] * 8 + [bspec] * 5,
        out_specs=[pspec] * 6,
        out_shape=[jax.ShapeDtypeStruct((NAP, 128), jnp.float32)] * 6,
    )(dg0p, dg1p, y1, y2, y3, y4, y5, isd_p, b1, b2, b3, b4, b5)


def _midA_tc(pacc0, pacc1, dinv_p, sp, isd_p, W2p2d_big, b2p2d_p):
    def body(p0_r, p1_r, dinv_r, sp_r, isd_r, wp2d, bp2d, p2d2_o):
        dinv = dinv_r[...]
        mp = 1.0 - isd_r[...]
        hp = jnp.maximum(dinv * (p0_r[...] + p1_r[...]) + sp_r[...], 0.0)
        y = jnp.dot(hp, wp2d[...], preferred_element_type=jnp.float32)
        p2d2_o[...] = (y + bp2d[...]) * mp * dinv

    pspec = _pk_spec()
    return pl.pallas_call(
        body,
        grid=(4,),
        in_specs=[pspec, pspec, pspec, pspec, pspec,
                  _full_spec((128, 128)), _full_spec((1, 128))],
        out_specs=pspec,
        out_shape=jax.ShapeDtypeStruct((NAP, 128), jnp.float32),
    )(pacc0, pacc1, dinv_p, sp, isd_p, W2p2d_big, b2p2d_p)


def _midB_tc(dacc0, dacc1, dinv_p, sd, isd_p, W2sd_big, b2sd_p):
    def body(d0_r, d1_r, dinv_r, sd_r, isd_r, wsd, bsd, sd2_o):
        dinv = dinv_r[...]
        md = isd_r[...]
        hd = jnp.maximum(dinv * (d0_r[...] + d1_r[...]) + sd_r[...], 0.0)
        y = jnp.dot(hd, wsd[...], preferred_element_type=jnp.float32)
        sd2_o[...] = (y + bsd[...]) * md

    pspec = _pk_spec()
    return pl.pallas_call(
        body,
        grid=(4,),
        in_specs=[pspec, pspec, pspec, pspec, pspec,
                  _full_spec((128, 128)), _full_spec((1, 128))],
        out_specs=pspec,
        out_shape=jax.ShapeDtypeStruct((NAP, 128), jnp.float32),
    )(dacc0, dacc1, dinv_p, sd, isd_p, W2sd_big, b2sd_p)


def _head_tc(ga0, ga1, gdv, gs2, clf, predictor, nb):
    def body(a0_r, a1_r, dv_r, s2_r, cl_r, p_r, out_r):
        hd = dv_r[...] * (a0_r[...] + a1_r[...]) + s2_r[...]
        h1 = hd[0:nb, :]
        h2 = hd[nb:2 * nb, :]
        cl = cl_r[...]
        acc = jnp.zeros((nb, 1), jnp.float32)
        for c in range(4):
            t = jnp.dot(h1, p_r[c], preferred_element_type=jnp.float32)
            s = jnp.sum(t * h2, axis=1, keepdims=True)
            acc = acc + jnp.where(cl == c, s, 0.0)
        out_r[...] = acc

    gspec = _full_spec((2 * nb, H))
    return pl.pallas_call(
        body,
        grid=(1,),
        in_specs=[gspec, gspec, gspec, gspec, _full_spec((nb, 1)),
                  _full_spec((4, H, H))],
        out_specs=_full_spec((nb, 1)),
        out_shape=jax.ShapeDtypeStruct((nb, 1), jnp.float32),
    )(ga0, ga1, gdv, gs2, clf, predictor)


def kernel(x_drugs, x_prots, is_drug, ppi_edge_idx, dpi_edge_idx, drug_pairs,
           cell_lines, predictor,
           W1_d2p, b1_d2p, W1_p2d, b1_p2d, W1_p2p, b1_p2p, W1_sd, b1_sd,
           W1_sp, b1_sp,
           W2_d2p, b2_d2p, W2_p2d, b2_p2d, W2_p2p, b2_p2p, W2_sd, b2_sd,
           W2_sp, b2_sp):
    ppi = ppi_edge_idx.astype(jnp.int32)
    dpi = dpi_edge_idx.astype(jnp.int32)

    dpi_s, dpi_d, dpi_g = _pad_edges(dpi[0], dpi[1])
    pdi_s, pdi_d, pdi_g = _pad_edges(dpi[1], dpi[0])
    ppi_s, ppi_d, ppi_g = _pad_edges(ppi[0], ppi[1])

    dg0, dg1 = _make_deg([dpi_g, pdi_g, ppi_g])(dpi_d, pdi_d, ppi_d)

    xd4 = jnp.pad(x_drugs, ((0, NACC - N), (0, 0))).reshape(NAP, 8, 1, 128)
    xp4 = jnp.pad(x_prots, ((0, NACC - N), (0, 0))).reshape(NAP, 8, 1, 128)
    S8 = jnp.eye(128, dtype=jnp.float32).reshape(8, H, 128)

    def w8(w):
        return jnp.einsum('kj,ajl->akl', w, S8)

    y1, y2, y3, y4, y5 = _preA_tc(
        xd4, xp4, w8(W1_d2p), w8(W1_p2d), w8(W1_p2p), w8(W1_sd), w8(W1_sp))

    isd_p = jnp.pad(jnp.repeat(is_drug, H),
                    (0, (NACC - N) * H)).reshape(NAP, 128)

    def bt(b):
        return jnp.tile(b, 8).reshape(1, 128)

    def quarter(a):
        return a.reshape(NAP, 128)

    d2p_s, p2d_s, p2p_s, sd, sp, dinv_p = _preB_tc(
        quarter(dg0), quarter(dg1), y1, y2, y3, y4, y5, isd_p,
        bt(b1_d2p), bt(b1_p2d), bt(b1_p2p), bt(b1_sd), bt(b1_sp))

    prop2 = _make_propagate([pdi_g])
    prop1 = _make_propagate([dpi_g, ppi_g])

    def lin(a):
        return a.reshape(NACC, H)

    pacc0, pacc1 = prop1(dpi_s, dpi_d, lin(d2p_s), ppi_s, ppi_d, lin(p2p_s))
    dacc0, dacc1 = prop2(pdi_s, pdi_d, lin(p2d_s))

    eye8 = jnp.eye(8, dtype=jnp.float32)
    p2d2_s = _midA_tc(quarter(pacc0), quarter(pacc1), dinv_p, sp, isd_p,
                      jnp.kron(eye8, W2_p2d), bt(b2_p2d))
    sd2 = _midB_tc(quarter(dacc0), quarter(dacc1), dinv_p, sd, isd_p,
                   jnp.kron(eye8, W2_sd), bt(b2_sd))

    acc20, acc21 = prop2(pdi_s, pdi_d, lin(p2d2_s))

    nb = drug_pairs.shape[0]
    pairs_m = jnp.concatenate(
        [drug_pairs[:, 0].astype(jnp.int32),
         drug_pairs[:, 1].astype(jnp.int32)]).reshape(64, 128)
    ga0, ga1, gdv, gs2 = _pairs_gather(pairs_m, acc20, acc21,
                                       lin(dinv_p), lin(sd2))

    clf = cell_lines.astype(jnp.float32).reshape(nb, 1)
    out = _head_tc(ga0.reshape(2 * nb, H), ga1.reshape(2 * nb, H),
                   gdv.reshape(2 * nb, H), gs2.reshape(2 * nb, H),
                   clf, predictor, nb)
    return out.reshape(nb)


# speedup vs baseline: 1.1051x; 1.0053x over previous
import functools

import jax
import jax.numpy as jnp
from jax import lax
from jax.experimental import pallas as pl
from jax.experimental.pallas import tpu as pltpu
from jax.experimental.pallas import tpu_sc as plsc

N = 100000
H = 16
NACC = 100096
STRIPE = NACC // 16
ZCH = STRIPE // 8
ZCH2 = STRIPE // 16
NC, NS, LN = 2, 16, 16
W = NC * NS
KJ = 8
KJP = 4
GROUP_E = W * 128 * KJ
GROUP_P = W * 128 * KJP
CORE0_FRAC = 0.6

BN = 2000
GRID = N // BN

_MESH = plsc.VectorSubcoreMesh(core_axis_name="c", subcore_axis_name="s")
_SC_PARAMS = pltpu.CompilerParams(use_tc_tiling_on_sc=False)


def _pad_edges(src, dst):
    e = src.shape[0]
    e2 = ((e + GROUP_P - 1) // GROUP_P) * GROUP_P
    pad = e2 - e
    src = jnp.concatenate([src, jnp.zeros((pad,), jnp.int32)])
    dst = jnp.concatenate([dst, jnp.full((pad,), DUMMY, jnp.int32)])
    tp = e2 // (16 * KJP * 128)
    g0 = 2 * int(round(tp * CORE0_FRAC / 2))
    g0 = min(max(g0, 2), tp - 2)
    return src.reshape(-1, 128), dst.reshape(-1, 128), (g0, tp - g0)


def _make_propagate(groups_list):
    nsets = len(groups_list)

    def body(*refs):
        ins = refs[: 3 * nsets]
        out0, out1 = refs[3 * nsets: 3 * nsets + 2]
        (sidx0, didx0, rbuf0, sidx1, didx1, rbuf1, zbuf, acc,
         sem0, sem1) = refs[3 * nsets + 2:]
        c = lax.axis_index("c")
        s = lax.axis_index("s")
        wid = s * NC + c

        def zb(i, carry):
            zbuf[i] = jnp.zeros((H,), jnp.float32)
            return carry

        lax.fori_loop(0, ZCH2, zb, 0)
        for k in range(16):
            pltpu.sync_copy(zbuf, acc.at[pl.ds(s * STRIPE + k * ZCH2, ZCH2)])
        plsc.subcore_barrier()

        for t in range(nsets):
            srcm, dstm, xh = ins[3 * t: 3 * t + 3]
            G0, G1 = groups_list[t]
            Gc = jnp.where(c == 0, G0, G1)
            wbase = jnp.where(c == 0, s * (G0 * KJP),
                              16 * G0 * KJP + s * (G1 * KJP))

            def fire(roff, sidx, didx, rbuf, sem, srcm=srcm, dstm=dstm,
                     xh=xh):
                pltpu.sync_copy(srcm.at[pl.ds(roff, KJP)], sidx)
                pltpu.sync_copy(dstm.at[pl.ds(roff, KJP)], didx)
                for j in range(KJP):
                    pltpu.async_copy(xh.at[sidx.at[j]], rbuf.at[j], sem)

            def wait_g(sidx, rbuf, sem, xh=xh):
                for j in range(KJP):
                    pltpu.make_async_copy(
                        xh.at[sidx.at[j]], rbuf.at[j], sem).wait()

            def scat(didx, rbuf):
                for j in range(KJP):
                    pltpu.sync_copy(rbuf.at[j], acc.at[didx.at[j]], add=True)

            fire(wbase, sidx0, didx0, rbuf0, sem0)

            def dbl(i, carry, wbase=wbase, Gc=Gc, fire=fire, wait_g=wait_g,
                    scat=scat):
                g0 = 2 * i
                fire(wbase + (g0 + 1) * KJP, sidx1, didx1, rbuf1, sem1)
                wait_g(sidx0, rbuf0, sem0)
                scat(didx0, rbuf0)
                fire(wbase + jnp.minimum(g0 + 2, Gc - 1) * KJP,
                     sidx0, didx0, rbuf0, sem0)
                wait_g(sidx1, rbuf1, sem1)
                scat(didx1, rbuf1)
                return carry

            lax.fori_loop(0, Gc // 2, dbl, 0)
            wait_g(sidx0, rbuf0, sem0)

        plsc.subcore_barrier()

        @pl.when(c == 0)
        def _():
            pltpu.sync_copy(acc.at[pl.ds(s * STRIPE, STRIPE)],
                            out0.at[pl.ds(s * STRIPE, STRIPE)])

        @pl.when(c == 1)
        def _():
            pltpu.sync_copy(acc.at[pl.ds(s * STRIPE, STRIPE)],
                            out1.at[pl.ds(s * STRIPE, STRIPE)])

    return pl.kernel(
        body,
        out_type=[jax.ShapeDtypeStruct((NACC, H), jnp.float32)] * 2,
        mesh=_MESH,
        compiler_params=_SC_PARAMS,
        scratch_types=[
            pltpu.VMEM((KJP, 128), jnp.int32),
            pltpu.VMEM((KJP, 128), jnp.int32),
            pltpu.VMEM((KJP, 128, H), jnp.float32),
            pltpu.VMEM((KJP, 128), jnp.int32),
            pltpu.VMEM((KJP, 128), jnp.int32),
            pltpu.VMEM((KJP, 128, H), jnp.float32),
            pltpu.VMEM((ZCH2, H), jnp.float32),
            pltpu.VMEM_SHARED((NACC, H), jnp.float32),
            pltpu.SemaphoreType.DMA,
            pltpu.SemaphoreType.DMA,
        ],
    )


def _make_deg(groups_list):
    nsets = len(groups_list)

    def body(*refs):
        ins = refs[:nsets]
        out0, out1 = refs[nsets: nsets + 2]
        cidx, ones_v, degv, ebuf, deg_sh = refs[nsets + 2:]
        c = lax.axis_index("c")
        s = lax.axis_index("s")
        for k in range(8):
            ones_v[pl.ds(k * LN, LN)] = jnp.ones((LN,), jnp.float32)

        def zb(i, carry):
            degv[pl.ds(i * LN, LN)] = jnp.zeros((LN,), jnp.float32)
            return carry

        lax.fori_loop(0, STRIPE // LN, zb, 0)
        pltpu.sync_copy(degv, deg_sh.at[pl.ds(s * STRIPE, STRIPE)])
        plsc.subcore_barrier()

        for t in range(nsets):
            dstm = ins[t]
            G0, G1 = groups_list[t]
            Gc = jnp.where(c == 0, G0, G1)
            wbase = jnp.where(c == 0, s * (G0 * KJP),
                              16 * G0 * KJP + s * (G1 * KJP))

            def grp(g, carry, dstm=dstm, wbase=wbase):
                roff = wbase + g * KJP
                pltpu.sync_copy(dstm.at[pl.ds(roff, KJP)], cidx)
                for j in range(KJP):
                    pltpu.sync_copy(ones_v, deg_sh.at[cidx.at[j]], add=True)
                return carry

            lax.fori_loop(0, Gc, grp, 0)

        plsc.subcore_barrier()

        pltpu.sync_copy(deg_sh.at[pl.ds(s * STRIPE, STRIPE)], degv)

        def exp(i, carry):
            v = degv[pl.ds(i * LN, LN)]
            for l in range(LN):
                ebuf[i * LN + l] = jnp.full((H,), v[l])
            return carry

        lax.fori_loop(0, STRIPE // LN, exp, 0)

        @pl.when(c == 0)
        def _():
            pltpu.sync_copy(ebuf, out0.at[pl.ds(s * STRIPE, STRIPE)])

        @pl.when(c == 1)
        def _():
            pltpu.sync_copy(ebuf, out1.at[pl.ds(s * STRIPE, STRIPE)])

    return pl.kernel(
        body,
        out_type=[jax.ShapeDtypeStruct((NACC, H), jnp.float32)] * 2,
        mesh=_MESH,
        compiler_params=_SC_PARAMS,
        scratch_types=[
            pltpu.VMEM((KJP, 128), jnp.int32),
            pltpu.VMEM((128,), jnp.float32),
            pltpu.VMEM((STRIPE,), jnp.float32),
            pltpu.VMEM((STRIPE, H), jnp.float32),
            pltpu.VMEM_SHARED((NACC,), jnp.float32),
        ],
    )


def _pairs_gather(pairs_m, a0, a1, dv, s2):
    def body(pm, t0, t1, t2, t3, o0, o1, o2, o3, pidx, gbuf, sem):
        c = lax.axis_index("c")
        s = lax.axis_index("s")
        wid = s * NC + c
        pltpu.sync_copy(pm.at[pl.ds(wid * 2, 2)], pidx)
        for th, out in ((t0, o0), (t1, o1), (t2, o2), (t3, o3)):
            descs = [
                pltpu.async_copy(th.at[pidx.at[j]], gbuf.at[j], sem)
                for j in range(2)
            ]
            for d in descs:
                d.wait()
            pltpu.sync_copy(gbuf, out.at[pl.ds(wid * 2, 2)])

    f = pl.kernel(
        body,
        out_type=[jax.ShapeDtypeStruct((64, 128, H), jnp.float32)] * 4,
        mesh=_MESH,
        compiler_params=_SC_PARAMS,
        scratch_types=[
            pltpu.VMEM((2, 128), jnp.int32),
            pltpu.VMEM((2, 128, H), jnp.float32),
            pltpu.SemaphoreType.DMA,
        ],
    )
    return f(pairs_m, a0, a1, dv, s2)


NAP = NACC // 8
BR = NAP // 4
DUMMY = N


def _pk_spec():
    return pl.BlockSpec((BR, 128), lambda i: (i, 0))


def _full_spec(shape):
    nd = len(shape)
    return pl.BlockSpec(shape, lambda i: (0,) * nd)


def _preA_tc(xd4, xp4, Wd2p8, Wp2d8, Wp2p8, Wsd8, Wsp8):
    def body(xd_r, xp_r, wd2p, wp2d, wp2p, wsd, wsp,
             d2p_o, p2d_o, p2p_o, sd_o, sp_o):
        a = pl.program_id(1)
        xd = xd_r[:, 0, 0, :]
        xp = xp_r[:, 0, 0, :]
        dot = functools.partial(jnp.dot, preferred_element_type=jnp.float32)
        outs = ((d2p_o, xd, wd2p), (p2d_o, xp, wp2d), (p2p_o, xp, wp2p),
                (sd_o, xd, wsd), (sp_o, xp, wsp))

        @pl.when(a == 0)
        def _():
            for o, x, w in outs:
                o[...] = dot(x, w[0])

        @pl.when(a != 0)
        def _():
            for o, x, w in outs:
                o[...] += dot(x, w[0])

    x4spec = pl.BlockSpec((BR, 1, 1, 128), lambda i, a: (i, a, 0, 0))
    wspec = pl.BlockSpec((1, 128, 128), lambda i, a: (a, 0, 0))
    pspec = pl.BlockSpec((BR, 128), lambda i, a: (i, 0))
    return pl.pallas_call(
        body,
        grid=(4, 8),
        in_specs=[x4spec, x4spec, wspec, wspec, wspec, wspec, wspec],
        out_specs=[pspec] * 5,
        out_shape=[jax.ShapeDtypeStruct((NAP, 128), jnp.float32)] * 5,
    )(xd4, xp4, Wd2p8, Wp2d8, Wp2p8, Wsd8, Wsp8)


def _preB_tc(dg0p, dg1p, y1, y2, y3, y4, y5, isd_p, b1, b2, b3, b4, b5):
    def body(d0_r, d1_r, y1_r, y2_r, y3_r, y4_r, y5_r, isd_r,
             b1_r, b2_r, b3_r, b4_r, b5_r,
             d2p_o, p2d_o, p2p_o, sd_o, sp_o, dinv_o):
        deg = d0_r[...] + d1_r[...]
        dinv = jnp.where(deg > 0, lax.rsqrt(jnp.maximum(deg, 1e-30)), 0.0)
        md = isd_r[...]
        mp = 1.0 - md
        d2p_o[...] = (y1_r[...] + b1_r[...]) * md * dinv
        p2d_o[...] = (y2_r[...] + b2_r[...]) * mp * dinv
        p2p_o[...] = (y3_r[...] + b3_r[...]) * mp * dinv
        sd_o[...] = (y4_r[...] + b4_r[...]) * md
        sp_o[...] = (y5_r[...] + b5_r[...]) * mp
        dinv_o[...] = dinv

    pspec = _pk_spec()
    bspec = _full_spec((1, 128))
    return pl.pallas_call(
        body,
        grid=(4,),
        in_specs=[pspec] * 8 + [bspec] * 5,
        out_specs=[pspec] * 6,
        out_shape=[jax.ShapeDtypeStruct((NAP, 128), jnp.float32)] * 6,
    )(dg0p, dg1p, y1, y2, y3, y4, y5, isd_p, b1, b2, b3, b4, b5)


def _midA_tc(pacc0, pacc1, dinv_p, sp, isd_p, W2p2d_big, b2p2d_p):
    def body(p0_r, p1_r, dinv_r, sp_r, isd_r, wp2d, bp2d, p2d2_o):
        dinv = dinv_r[...]
        mp = 1.0 - isd_r[...]
        hp = jnp.maximum(dinv * (p0_r[...] + p1_r[...]) + sp_r[...], 0.0)
        y = jnp.dot(hp, wp2d[...], preferred_element_type=jnp.float32)
        p2d2_o[...] = (y + bp2d[...]) * mp * dinv

    pspec = _pk_spec()
    return pl.pallas_call(
        body,
        grid=(4,),
        in_specs=[pspec, pspec, pspec, pspec, pspec,
                  _full_spec((128, 128)), _full_spec((1, 128))],
        out_specs=pspec,
        out_shape=jax.ShapeDtypeStruct((NAP, 128), jnp.float32),
    )(pacc0, pacc1, dinv_p, sp, isd_p, W2p2d_big, b2p2d_p)


def _midB_tc(dacc0, dacc1, dinv_p, sd, isd_p, W2sd_big, b2sd_p):
    def body(d0_r, d1_r, dinv_r, sd_r, isd_r, wsd, bsd, sd2_o):
        dinv = dinv_r[...]
        md = isd_r[...]
        hd = jnp.maximum(dinv * (d0_r[...] + d1_r[...]) + sd_r[...], 0.0)
        y = jnp.dot(hd, wsd[...], preferred_element_type=jnp.float32)
        sd2_o[...] = (y + bsd[...]) * md

    pspec = _pk_spec()
    return pl.pallas_call(
        body,
        grid=(4,),
        in_specs=[pspec, pspec, pspec, pspec, pspec,
                  _full_spec((128, 128)), _full_spec((1, 128))],
        out_specs=pspec,
        out_shape=jax.ShapeDtypeStruct((NAP, 128), jnp.float32),
    )(dacc0, dacc1, dinv_p, sd, isd_p, W2sd_big, b2sd_p)


def _head_tc(ga0, ga1, gdv, gs2, clf, predictor, nb):
    def body(a0_r, a1_r, dv_r, s2_r, cl_r, p_r, out_r):
        hd = dv_r[...] * (a0_r[...] + a1_r[...]) + s2_r[...]
        h1 = hd[0:nb, :]
        h2 = hd[nb:2 * nb, :]
        cl = cl_r[...]
        acc = jnp.zeros((nb, 1), jnp.float32)
        for c in range(4):
            t = jnp.dot(h1, p_r[c], preferred_element_type=jnp.float32)
            s = jnp.sum(t * h2, axis=1, keepdims=True)
            acc = acc + jnp.where(cl == c, s, 0.0)
        out_r[...] = acc

    gspec = _full_spec((2 * nb, H))
    return pl.pallas_call(
        body,
        grid=(1,),
        in_specs=[gspec, gspec, gspec, gspec, _full_spec((nb, 1)),
                  _full_spec((4, H, H))],
        out_specs=_full_spec((nb, 1)),
        out_shape=jax.ShapeDtypeStruct((nb, 1), jnp.float32),
    )(ga0, ga1, gdv, gs2, clf, predictor)


def kernel(x_drugs, x_prots, is_drug, ppi_edge_idx, dpi_edge_idx, drug_pairs,
           cell_lines, predictor,
           W1_d2p, b1_d2p, W1_p2d, b1_p2d, W1_p2p, b1_p2p, W1_sd, b1_sd,
           W1_sp, b1_sp,
           W2_d2p, b2_d2p, W2_p2d, b2_p2d, W2_p2p, b2_p2p, W2_sd, b2_sd,
           W2_sp, b2_sp):
    ppi = ppi_edge_idx.astype(jnp.int32)
    dpi = dpi_edge_idx.astype(jnp.int32)

    dpi_s, dpi_d, dpi_g = _pad_edges(dpi[0], dpi[1])
    pdi_s, pdi_d, pdi_g = _pad_edges(dpi[1], dpi[0])
    ppi_s, ppi_d, ppi_g = _pad_edges(ppi[0], ppi[1])

    dg0, dg1 = _make_deg([dpi_g, pdi_g, ppi_g])(dpi_d, pdi_d, ppi_d)

    xd4 = jnp.pad(x_drugs, ((0, NACC - N), (0, 0))).reshape(NAP, 8, 1, 128)
    xp4 = jnp.pad(x_prots, ((0, NACC - N), (0, 0))).reshape(NAP, 8, 1, 128)
    S8 = jnp.eye(128, dtype=jnp.float32).reshape(8, H, 128)

    def w8(w):
        return jnp.einsum('kj,ajl->akl', w, S8)

    y1, y2, y3, y4, y5 = _preA_tc(
        xd4, xp4, w8(W1_d2p), w8(W1_p2d), w8(W1_p2p), w8(W1_sd), w8(W1_sp))

    isd_p = jnp.pad(jnp.repeat(is_drug, H),
                    (0, (NACC - N) * H)).reshape(NAP, 128)

    def bt(b):
        return jnp.tile(b, 8).reshape(1, 128)

    def quarter(a):
        return a.reshape(NAP, 128)

    d2p_s, p2d_s, p2p_s, sd, sp, dinv_p = _preB_tc(
        quarter(dg0), quarter(dg1), y1, y2, y3, y4, y5, isd_p,
        bt(b1_d2p), bt(b1_p2d), bt(b1_p2p), bt(b1_sd), bt(b1_sp))

    prop2 = _make_propagate([pdi_g])
    prop1 = _make_propagate([dpi_g, ppi_g])

    def lin(a):
        return a.reshape(NACC, H)

    pacc0, pacc1 = prop1(dpi_s, dpi_d, lin(d2p_s), ppi_s, ppi_d, lin(p2p_s))
    dacc0, dacc1 = prop2(pdi_s, pdi_d, lin(p2d_s))

    eye8 = jnp.eye(8, dtype=jnp.float32)
    p2d2_s = _midA_tc(quarter(pacc0), quarter(pacc1), dinv_p, sp, isd_p,
                      jnp.kron(eye8, W2_p2d), bt(b2_p2d))
    sd2 = _midB_tc(quarter(dacc0), quarter(dacc1), dinv_p, sd, isd_p,
                   jnp.kron(eye8, W2_sd), bt(b2_sd))

    acc20, acc21 = prop2(pdi_s, pdi_d, lin(p2d2_s))

    nb = drug_pairs.shape[0]
    pairs_m = jnp.concatenate(
        [drug_pairs[:, 0].astype(jnp.int32),
         drug_pairs[:, 1].astype(jnp.int32)]).reshape(64, 128)
    ga0, ga1, gdv, gs2 = _pairs_gather(pairs_m, acc20, acc21,
                                       lin(dinv_p), lin(sd2))

    clf = cell_lines.astype(jnp.float32).reshape(nb, 1)
    out = _head_tc(ga0.reshape(2 * nb, H), ga1.reshape(2 * nb, H),
                   gdv.reshape(2 * nb, H), gs2.reshape(2 * nb, H),
                   clf, predictor, nb)
    return out.reshape(nb)


# speedup vs baseline: 1.2797x; 1.1580x over previous
import functools

import jax
import jax.numpy as jnp
from jax import lax
from jax.experimental import pallas as pl
from jax.experimental.pallas import tpu as pltpu
from jax.experimental.pallas import tpu_sc as plsc

N = 100000
H = 16
NACC = 100096
STRIPE = NACC // 16
ZCH = STRIPE // 8
ZCH2 = STRIPE // 16
NC, NS, LN = 2, 16, 16
W = NC * NS
KJ = 8
KJP = 4
GROUP_E = W * 128 * KJ
GROUP_P = W * 128 * KJP
CORE0_FRAC = 0.6

BN = 2000
GRID = N // BN

_MESH = plsc.VectorSubcoreMesh(core_axis_name="c", subcore_axis_name="s")
_SC_PARAMS = pltpu.CompilerParams(use_tc_tiling_on_sc=False)


def _pad_edges(src, dst):
    e = src.shape[0]
    e2 = ((e + GROUP_P - 1) // GROUP_P) * GROUP_P
    pad = e2 - e
    src = jnp.concatenate([src, jnp.zeros((pad,), jnp.int32)])
    dst = jnp.concatenate([dst, jnp.full((pad,), DUMMY, jnp.int32)])
    tp = e2 // (16 * KJP * 128)
    g0 = 2 * int(round(tp * CORE0_FRAC / 2))
    g0 = min(max(g0, 2), tp - 2)
    return src.reshape(-1, 128), dst.reshape(-1, 128), (g0, tp - g0)


def _make_propagate(groups_list):
    nsets = len(groups_list)

    def body(*refs):
        ins = refs[: 3 * nsets]
        out0, out1 = refs[3 * nsets: 3 * nsets + 2]
        (sidx0, didx0, rbuf0, sidx1, didx1, rbuf1, zbuf, acc,
         sem0, sem1) = refs[3 * nsets + 2:]
        c = lax.axis_index("c")
        s = lax.axis_index("s")
        wid = s * NC + c

        def zb(i, carry):
            zbuf[i] = jnp.zeros((H,), jnp.float32)
            return carry

        lax.fori_loop(0, ZCH2, zb, 0)
        for k in range(16):
            pltpu.sync_copy(zbuf, acc.at[pl.ds(s * STRIPE + k * ZCH2, ZCH2)])
        plsc.subcore_barrier()

        for t in range(nsets):
            srcm, dstm, xh = ins[3 * t: 3 * t + 3]
            G0, G1 = groups_list[t]
            Gc = jnp.where(c == 0, G0, G1)
            wbase = jnp.where(c == 0, s * (G0 * KJP),
                              16 * G0 * KJP + s * (G1 * KJP))

            def fire(roff, sidx, didx, rbuf, sem, srcm=srcm, dstm=dstm,
                     xh=xh):
                pltpu.sync_copy(srcm.at[pl.ds(roff, KJP)], sidx)
                pltpu.sync_copy(dstm.at[pl.ds(roff, KJP)], didx)
                for j in range(KJP):
                    pltpu.async_copy(xh.at[sidx.at[j]], rbuf.at[j], sem)

            def wait_g(sidx, rbuf, sem, xh=xh):
                for j in range(KJP):
                    pltpu.make_async_copy(
                        xh.at[sidx.at[j]], rbuf.at[j], sem).wait()

            def scat(didx, rbuf):
                for j in range(KJP):
                    pltpu.sync_copy(rbuf.at[j], acc.at[didx.at[j]], add=True)

            fire(wbase, sidx0, didx0, rbuf0, sem0)

            def dbl(i, carry, wbase=wbase, Gc=Gc, fire=fire, wait_g=wait_g,
                    scat=scat):
                g0 = 2 * i
                fire(wbase + (g0 + 1) * KJP, sidx1, didx1, rbuf1, sem1)
                wait_g(sidx0, rbuf0, sem0)
                scat(didx0, rbuf0)
                fire(wbase + jnp.minimum(g0 + 2, Gc - 1) * KJP,
                     sidx0, didx0, rbuf0, sem0)
                wait_g(sidx1, rbuf1, sem1)
                scat(didx1, rbuf1)
                return carry

            lax.fori_loop(0, Gc // 2, dbl, 0)
            wait_g(sidx0, rbuf0, sem0)

        plsc.subcore_barrier()

        @pl.when(c == 0)
        def _():
            pltpu.sync_copy(acc.at[pl.ds(s * STRIPE, STRIPE)],
                            out0.at[pl.ds(s * STRIPE, STRIPE)])

        @pl.when(c == 1)
        def _():
            pltpu.sync_copy(acc.at[pl.ds(s * STRIPE, STRIPE)],
                            out1.at[pl.ds(s * STRIPE, STRIPE)])

    return pl.kernel(
        body,
        out_type=[jax.ShapeDtypeStruct((NACC, H), jnp.float32)] * 2,
        mesh=_MESH,
        compiler_params=_SC_PARAMS,
        scratch_types=[
            pltpu.VMEM((KJP, 128), jnp.int32),
            pltpu.VMEM((KJP, 128), jnp.int32),
            pltpu.VMEM((KJP, 128, H), jnp.float32),
            pltpu.VMEM((KJP, 128), jnp.int32),
            pltpu.VMEM((KJP, 128), jnp.int32),
            pltpu.VMEM((KJP, 128, H), jnp.float32),
            pltpu.VMEM((ZCH2, H), jnp.float32),
            pltpu.VMEM_SHARED((NACC, H), jnp.float32),
            pltpu.SemaphoreType.DMA,
            pltpu.SemaphoreType.DMA,
        ],
    )


def _make_deg(groups_list):
    nsets = len(groups_list)

    def body(*refs):
        ins = refs[:nsets]
        out0, out1 = refs[nsets: nsets + 2]
        cidx, ones_v, degv, ebuf, deg_sh = refs[nsets + 2:]
        c = lax.axis_index("c")
        s = lax.axis_index("s")
        for k in range(8):
            ones_v[pl.ds(k * LN, LN)] = jnp.ones((LN,), jnp.float32)

        def zb(i, carry):
            degv[pl.ds(i * LN, LN)] = jnp.zeros((LN,), jnp.float32)
            return carry

        lax.fori_loop(0, STRIPE // LN, zb, 0)
        pltpu.sync_copy(degv, deg_sh.at[pl.ds(s * STRIPE, STRIPE)])
        plsc.subcore_barrier()

        for t in range(nsets):
            dstm = ins[t]
            G0, G1 = groups_list[t]
            Gc = jnp.where(c == 0, G0, G1)
            wbase = jnp.where(c == 0, s * (G0 * KJP),
                              16 * G0 * KJP + s * (G1 * KJP))

            def grp(g, carry, dstm=dstm, wbase=wbase):
                roff = wbase + g * KJP
                pltpu.sync_copy(dstm.at[pl.ds(roff, KJP)], cidx)
                for j in range(KJP):
                    pltpu.sync_copy(ones_v, deg_sh.at[cidx.at[j]], add=True)
                return carry

            lax.fori_loop(0, Gc, grp, 0)

        plsc.subcore_barrier()

        pltpu.sync_copy(deg_sh.at[pl.ds(s * STRIPE, STRIPE)], degv)

        def exp(i, carry):
            v = degv[pl.ds(i * LN, LN)]
            for l in range(LN):
                ebuf[i * LN + l] = jnp.full((H,), v[l])
            return carry

        lax.fori_loop(0, STRIPE // LN, exp, 0)

        @pl.when(c == 0)
        def _():
            pltpu.sync_copy(ebuf, out0.at[pl.ds(s * STRIPE, STRIPE)])

        @pl.when(c == 1)
        def _():
            pltpu.sync_copy(ebuf, out1.at[pl.ds(s * STRIPE, STRIPE)])

    return pl.kernel(
        body,
        out_type=[jax.ShapeDtypeStruct((NACC, H), jnp.float32)] * 2,
        mesh=_MESH,
        compiler_params=_SC_PARAMS,
        scratch_types=[
            pltpu.VMEM((KJP, 128), jnp.int32),
            pltpu.VMEM((128,), jnp.float32),
            pltpu.VMEM((STRIPE,), jnp.float32),
            pltpu.VMEM((STRIPE, H), jnp.float32),
            pltpu.VMEM_SHARED((NACC,), jnp.float32),
        ],
    )


def _pairs_gather(pairs_m, a0, a1, dv, s2):
    def body(pm, t0, t1, t2, t3, o0, o1, o2, o3, pidx, gbuf, sem):
        c = lax.axis_index("c")
        s = lax.axis_index("s")
        wid = s * NC + c
        pltpu.sync_copy(pm.at[pl.ds(wid * 2, 2)], pidx)
        for th, out in ((t0, o0), (t1, o1), (t2, o2), (t3, o3)):
            descs = [
                pltpu.async_copy(th.at[pidx.at[j]], gbuf.at[j], sem)
                for j in range(2)
            ]
            for d in descs:
                d.wait()
            pltpu.sync_copy(gbuf, out.at[pl.ds(wid * 2, 2)])

    f = pl.kernel(
        body,
        out_type=[jax.ShapeDtypeStruct((64, 128, H), jnp.float32)] * 4,
        mesh=_MESH,
        compiler_params=_SC_PARAMS,
        scratch_types=[
            pltpu.VMEM((2, 128), jnp.int32),
            pltpu.VMEM((2, 128, H), jnp.float32),
            pltpu.SemaphoreType.DMA,
        ],
    )
    return f(pairs_m, a0, a1, dv, s2)


NAP = NACC // 8
BR = NAP // 4
DUMMY = N


def _pk_spec():
    return pl.BlockSpec((BR, 128), lambda i: (i, 0))


def _full_spec(shape):
    nd = len(shape)
    return pl.BlockSpec(shape, lambda i: (0,) * nd)


def _preA_tc(xd4, xp4, Wd2p8, Wp2d8, Wp2p8, Wsd8, Wsp8):
    def body(xd_r, xp_r, wd2p, wp2d, wp2p, wsd, wsp,
             d2p_o, p2d_o, p2p_o, sd_o, sp_o):
        a = pl.program_id(1)
        xd = xd_r[:, 0, 0, :]
        xp = xp_r[:, 0, 0, :]
        dot = functools.partial(jnp.dot, preferred_element_type=jnp.float32)
        outs = ((d2p_o, xd, wd2p), (p2d_o, xp, wp2d), (p2p_o, xp, wp2p),
                (sd_o, xd, wsd), (sp_o, xp, wsp))

        @pl.when(a == 0)
        def _():
            for o, x, w in outs:
                o[...] = dot(x, w[0])

        @pl.when(a != 0)
        def _():
            for o, x, w in outs:
                o[...] += dot(x, w[0])

    x4spec = pl.BlockSpec((BR, 1, 1, 128), lambda i, a: (i, a, 0, 0))
    wspec = pl.BlockSpec((1, 128, 128), lambda i, a: (a, 0, 0))
    pspec = pl.BlockSpec((BR, 128), lambda i, a: (i, 0))
    return pl.pallas_call(
        body,
        grid=(4, 8),
        in_specs=[x4spec, x4spec, wspec, wspec, wspec, wspec, wspec],
        out_specs=[pspec] * 5,
        out_shape=[jax.ShapeDtypeStruct((NAP, 128), jnp.float32)] * 5,
    )(xd4, xp4, Wd2p8, Wp2d8, Wp2p8, Wsd8, Wsp8)


def _preB_tc(dg0p, dg1p, y1, y2, y3, y4, y5, isd8, rmat, b1, b2, b3, b4, b5):
    def body(d0_r, d1_r, y1_r, y2_r, y3_r, y4_r, y5_r, isd_r, rm_r,
             b1_r, b2_r, b3_r, b4_r, b5_r,
             d2p_o, p2d_o, p2p_o, sd_o, sp_o, dinv_o):
        deg = d0_r[...] + d1_r[...]
        dinv = jnp.where(deg > 0, lax.rsqrt(jnp.maximum(deg, 1e-30)), 0.0)
        md = jnp.dot(isd_r[...], rm_r[...], preferred_element_type=jnp.float32)
        mp = 1.0 - md
        d2p_o[...] = (y1_r[...] + b1_r[...]) * md * dinv
        p2d_o[...] = (y2_r[...] + b2_r[...]) * mp * dinv
        p2p_o[...] = (y3_r[...] + b3_r[...]) * mp * dinv
        sd_o[...] = (y4_r[...] + b4_r[...]) * md
        sp_o[...] = (y5_r[...] + b5_r[...]) * mp
        dinv_o[...] = dinv

    pspec = _pk_spec()
    bspec = _full_spec((1, 128))
    i8spec = pl.BlockSpec((BR, 8), lambda i: (i, 0))
    return pl.pallas_call(
        body,
        grid=(4,),
        in_specs=[pspec] * 7 + [i8spec, _full_spec((8, 128))] + [bspec] * 5,
        out_specs=[pspec] * 6,
        out_shape=[jax.ShapeDtypeStruct((NAP, 128), jnp.float32)] * 6,
    )(dg0p, dg1p, y1, y2, y3, y4, y5, isd8, rmat, b1, b2, b3, b4, b5)


def _midA_tc(pacc0, pacc1, dinv_p, sp, isd8, rmat, W2p2d_big, b2p2d_p):
    def body(p0_r, p1_r, dinv_r, sp_r, isd_r, rm_r, wp2d, bp2d, p2d2_o):
        dinv = dinv_r[...]
        mp = 1.0 - jnp.dot(isd_r[...], rm_r[...],
                           preferred_element_type=jnp.float32)
        hp = jnp.maximum(dinv * (p0_r[...] + p1_r[...]) + sp_r[...], 0.0)
        y = jnp.dot(hp, wp2d[...], preferred_element_type=jnp.float32)
        p2d2_o[...] = (y + bp2d[...]) * mp * dinv

    pspec = _pk_spec()
    i8spec = pl.BlockSpec((BR, 8), lambda i: (i, 0))
    return pl.pallas_call(
        body,
        grid=(4,),
        in_specs=[pspec, pspec, pspec, pspec, i8spec, _full_spec((8, 128)),
                  _full_spec((128, 128)), _full_spec((1, 128))],
        out_specs=pspec,
        out_shape=jax.ShapeDtypeStruct((NAP, 128), jnp.float32),
    )(pacc0, pacc1, dinv_p, sp, isd8, rmat, W2p2d_big, b2p2d_p)


def _midB_tc(dacc0, dacc1, dinv_p, sd, isd8, rmat, W2sd_big, b2sd_p):
    def body(d0_r, d1_r, dinv_r, sd_r, isd_r, rm_r, wsd, bsd, sd2_o):
        dinv = dinv_r[...]
        md = jnp.dot(isd_r[...], rm_r[...],
                     preferred_element_type=jnp.float32)
        hd = jnp.maximum(dinv * (d0_r[...] + d1_r[...]) + sd_r[...], 0.0)
        y = jnp.dot(hd, wsd[...], preferred_element_type=jnp.float32)
        sd2_o[...] = (y + bsd[...]) * md

    pspec = _pk_spec()
    i8spec = pl.BlockSpec((BR, 8), lambda i: (i, 0))
    return pl.pallas_call(
        body,
        grid=(4,),
        in_specs=[pspec, pspec, pspec, pspec, i8spec, _full_spec((8, 128)),
                  _full_spec((128, 128)), _full_spec((1, 128))],
        out_specs=pspec,
        out_shape=jax.ShapeDtypeStruct((NAP, 128), jnp.float32),
    )(dacc0, dacc1, dinv_p, sd, isd8, rmat, W2sd_big, b2sd_p)


def _head_tc(ga0, ga1, gdv, gs2, clf, predictor, nb):
    def body(a0_r, a1_r, dv_r, s2_r, cl_r, p_r, out_r):
        hd = dv_r[...] * (a0_r[...] + a1_r[...]) + s2_r[...]
        h1 = hd[0:nb, :]
        h2 = hd[nb:2 * nb, :]
        cl = cl_r[...]
        acc = jnp.zeros((nb, 1), jnp.float32)
        for c in range(4):
            t = jnp.dot(h1, p_r[c], preferred_element_type=jnp.float32)
            s = jnp.sum(t * h2, axis=1, keepdims=True)
            acc = acc + jnp.where(cl == c, s, 0.0)
        out_r[...] = acc

    gspec = _full_spec((2 * nb, H))
    return pl.pallas_call(
        body,
        grid=(1,),
        in_specs=[gspec, gspec, gspec, gspec, _full_spec((nb, 1)),
                  _full_spec((4, H, H))],
        out_specs=_full_spec((nb, 1)),
        out_shape=jax.ShapeDtypeStruct((nb, 1), jnp.float32),
    )(ga0, ga1, gdv, gs2, clf, predictor)


def kernel(x_drugs, x_prots, is_drug, ppi_edge_idx, dpi_edge_idx, drug_pairs,
           cell_lines, predictor,
           W1_d2p, b1_d2p, W1_p2d, b1_p2d, W1_p2p, b1_p2p, W1_sd, b1_sd,
           W1_sp, b1_sp,
           W2_d2p, b2_d2p, W2_p2d, b2_p2d, W2_p2p, b2_p2p, W2_sd, b2_sd,
           W2_sp, b2_sp):
    ppi = ppi_edge_idx.astype(jnp.int32)
    dpi = dpi_edge_idx.astype(jnp.int32)

    dpi_s, dpi_d, dpi_g = _pad_edges(dpi[0], dpi[1])
    pdi_s, pdi_d, pdi_g = _pad_edges(dpi[1], dpi[0])
    ppi_s, ppi_d, ppi_g = _pad_edges(ppi[0], ppi[1])

    dg0, dg1 = _make_deg([dpi_g, pdi_g, ppi_g])(dpi_d, pdi_d, ppi_d)

    xd4 = x_drugs.reshape(N // 8, 8, 1, 128)
    xp4 = x_prots.reshape(N // 8, 8, 1, 128)
    S8 = jnp.eye(128, dtype=jnp.float32).reshape(8, H, 128)

    def w8(w):
        return jnp.einsum('kj,ajl->akl', w, S8)

    y1, y2, y3, y4, y5 = _preA_tc(
        xd4, xp4, w8(W1_d2p), w8(W1_p2d), w8(W1_p2p), w8(W1_sd), w8(W1_sp))

    isd8 = jnp.pad(is_drug, (0, NACC - N)).reshape(NAP, 8)
    rmat = jnp.repeat(jnp.eye(8, dtype=jnp.float32), H, axis=1)

    def bt(b):
        return jnp.tile(b, 8).reshape(1, 128)

    def quarter(a):
        return a.reshape(NAP, 128)

    d2p_s, p2d_s, p2p_s, sd, sp, dinv_p = _preB_tc(
        quarter(dg0), quarter(dg1), y1, y2, y3, y4, y5, isd8, rmat,
        bt(b1_d2p), bt(b1_p2d), bt(b1_p2p), bt(b1_sd), bt(b1_sp))

    prop2 = _make_propagate([pdi_g])
    prop1 = _make_propagate([dpi_g, ppi_g])

    def lin(a):
        return a.reshape(NACC, H)

    pacc0, pacc1 = prop1(dpi_s, dpi_d, lin(d2p_s), ppi_s, ppi_d, lin(p2p_s))
    dacc0, dacc1 = prop2(pdi_s, pdi_d, lin(p2d_s))

    eye8 = jnp.eye(8, dtype=jnp.float32)
    p2d2_s = _midA_tc(quarter(pacc0), quarter(pacc1), dinv_p, sp, isd8,
                      rmat, jnp.kron(eye8, W2_p2d), bt(b2_p2d))
    sd2 = _midB_tc(quarter(dacc0), quarter(dacc1), dinv_p, sd, isd8,
                   rmat, jnp.kron(eye8, W2_sd), bt(b2_sd))

    acc20, acc21 = prop2(pdi_s, pdi_d, lin(p2d2_s))

    nb = drug_pairs.shape[0]
    pairs_m = jnp.concatenate(
        [drug_pairs[:, 0].astype(jnp.int32),
         drug_pairs[:, 1].astype(jnp.int32)]).reshape(64, 128)
    ga0, ga1, gdv, gs2 = _pairs_gather(pairs_m, acc20, acc21,
                                       lin(dinv_p), lin(sd2))

    clf = cell_lines.astype(jnp.float32).reshape(nb, 1)
    out = _head_tc(ga0.reshape(2 * nb, H), ga1.reshape(2 * nb, H),
                   gdv.reshape(2 * nb, H), gs2.reshape(2 * nb, H),
                   clf, predictor, nb)
    return out.reshape(nb)
